# initial kernel scaffold (unmeasured)
import jax
import jax.numpy as jnp
from jax import lax
from jax.experimental import pallas as pl
from jax.experimental.pallas import tpu as pltpu

N_DEV = 4
WINDOW = 128
NGLOB = 32
SCALE = 0.125
NEG = -1e9


def kernel(x, Wq, K_ext, V_ext, Wo):
    B, S_loc, E = x.shape
    _, _, HQ, DH = K_ext.shape
    DM = HQ * DH

    k_t = jnp.transpose(K_ext, (0, 2, 1, 3))
    v_t = jnp.transpose(V_ext, (0, 2, 1, 3))

    def body(x_ref, wq_ref, k_ref, v_ref, wo_ref, out_ref,
             k_full, v_full, k_send, k_recv, v_send, v_recv):
        my = lax.axis_index("i")
        left = lax.rem(my + N_DEV - 1, N_DEV)
        right = lax.rem(my + 1, N_DEV)

        barrier_sem = pltpu.get_barrier_semaphore()
        for nbr in (left, right):
            pl.semaphore_signal(barrier_sem, inc=1, device_id=(nbr,),
                                device_id_type=pl.DeviceIdType.MESH)
        pl.semaphore_wait(barrier_sem, 2)

        k_full[0] = k_ref[...]
        v_full[0] = v_ref[...]

        for h in range(N_DEV - 1):
            k_rdma = pltpu.make_async_remote_copy(
                src_ref=k_full.at[h], dst_ref=k_full.at[h + 1],
                send_sem=k_send.at[h], recv_sem=k_recv.at[h],
                device_id=(right,), device_id_type=pl.DeviceIdType.MESH)
            v_rdma = pltpu.make_async_remote_copy(
                src_ref=v_full.at[h], dst_ref=v_full.at[h + 1],
                send_sem=v_send.at[h], recv_sem=v_recv.at[h],
                device_id=(right,), device_id_type=pl.DeviceIdType.MESH)
            k_rdma.start()
            v_rdma.start()
            k_rdma.wait()
            v_rdma.wait()

        qi = my * S_loc + lax.broadcasted_iota(jnp.int32, (S_loc, S_loc), 0)
        kj_base = lax.broadcasted_iota(jnp.int32, (S_loc, S_loc), 1)
        masks = []
        for r in range(N_DEV):
            c = lax.rem(my - r + N_DEV, N_DEV)
            kj = c * S_loc + kj_base
            m = (jnp.abs(qi - kj) <= WINDOW) | (kj < NGLOB) | (qi < NGLOB)
            masks.append(m)

        for b in range(B):
            Qb = jnp.dot(x_ref[b], wq_ref[...],
                         preferred_element_type=jnp.float32)
            ctx_cols = []
            for hh in range(HQ):
                q = Qb[:, hh * DH:(hh + 1) * DH]
                s_parts = []
                for r in range(N_DEV):
                    kk = k_full[r, b, hh]
                    s = lax.dot_general(
                        q, kk, (((1,), (1,)), ((), ())),
                        preferred_element_type=jnp.float32) * SCALE
                    s_parts.append(jnp.where(masks[r], s, NEG))
                sc = jnp.concatenate(s_parts, axis=1)
                mmax = jnp.max(sc, axis=1, keepdims=True)
                w = jnp.exp(sc - mmax)
                w = w / jnp.sum(w, axis=1, keepdims=True)
                ctx = jnp.zeros((S_loc, DH), jnp.float32)
                for r in range(N_DEV):
                    ctx = ctx + jnp.dot(
                        w[:, r * S_loc:(r + 1) * S_loc], v_full[r, b, hh],
                        preferred_element_type=jnp.float32)
                ctx_cols.append(ctx)
            ctx_b = jnp.concatenate(ctx_cols, axis=1)
            out_ref[b] = jnp.dot(ctx_b, wo_ref[...],
                                 preferred_element_type=jnp.float32)

    return pl.pallas_call(
        body,
        out_shape=jax.ShapeDtypeStruct((B, S_loc, E), jnp.float32),
        in_specs=[pl.BlockSpec(memory_space=pltpu.VMEM)] * 5,
        out_specs=pl.BlockSpec(memory_space=pltpu.VMEM),
        scratch_shapes=[
            pltpu.VMEM((N_DEV, B, HQ, S_loc, DH), jnp.float32),
            pltpu.VMEM((N_DEV, B, HQ, S_loc, DH), jnp.float32),
            pltpu.SemaphoreType.DMA((N_DEV - 1,)),
            pltpu.SemaphoreType.DMA((N_DEV - 1,)),
            pltpu.SemaphoreType.DMA((N_DEV - 1,)),
            pltpu.SemaphoreType.DMA((N_DEV - 1,)),
        ],
        compiler_params=pltpu.CompilerParams(collective_id=0),
    )(x, Wq, k_t, v_t, Wo)


# baseline (device time: 339407 ns/iter reference)
import jax
import jax.numpy as jnp
from jax import lax
from jax.experimental import pallas as pl
from jax.experimental.pallas import tpu as pltpu

N_DEV = 4
WINDOW = 128
NGLOB = 32
SCALE = 0.125
NEG = -1e9


def kernel(x, Wq, K_ext, V_ext, Wo):
    B, S_loc, E = x.shape
    _, _, HQ, DH = K_ext.shape
    DM = HQ * DH

    k_t = jnp.transpose(K_ext, (0, 2, 1, 3))
    v_t = jnp.transpose(V_ext, (0, 2, 1, 3))

    def body(x_ref, wq_ref, k_ref, v_ref, wo_ref, out_ref,
             k_full, v_full, k_send, k_recv, v_send, v_recv):
        my = lax.axis_index("i")
        left = lax.rem(my + N_DEV - 1, N_DEV)
        right = lax.rem(my + 1, N_DEV)

        barrier_sem = pltpu.get_barrier_semaphore()
        for nbr in (left, right):
            pl.semaphore_signal(barrier_sem, inc=1, device_id=(nbr,),
                                device_id_type=pl.DeviceIdType.MESH)
        pl.semaphore_wait(barrier_sem, 2)

        k_full[0] = k_ref[...]
        v_full[0] = v_ref[...]

        for h in range(N_DEV - 1):
            k_rdma = pltpu.make_async_remote_copy(
                src_ref=k_full.at[h], dst_ref=k_full.at[h + 1],
                send_sem=k_send.at[h], recv_sem=k_recv.at[h],
                device_id=(right,), device_id_type=pl.DeviceIdType.MESH)
            v_rdma = pltpu.make_async_remote_copy(
                src_ref=v_full.at[h], dst_ref=v_full.at[h + 1],
                send_sem=v_send.at[h], recv_sem=v_recv.at[h],
                device_id=(right,), device_id_type=pl.DeviceIdType.MESH)
            k_rdma.start()
            v_rdma.start()
            k_rdma.wait()
            v_rdma.wait()

        qi = my * S_loc + lax.broadcasted_iota(jnp.int32, (S_loc, S_loc), 0)
        kj_base = lax.broadcasted_iota(jnp.int32, (S_loc, S_loc), 1)
        masks = []
        for r in range(N_DEV):
            c = lax.rem(my - r + N_DEV, N_DEV)
            kj = c * S_loc + kj_base
            m = (jnp.abs(qi - kj) <= WINDOW) | (kj < NGLOB) | (qi < NGLOB)
            masks.append(m)

        for b in range(B):
            Qb = jnp.dot(x_ref[b], wq_ref[...],
                         preferred_element_type=jnp.float32)
            ctx_cols = []
            for hh in range(HQ):
                q = Qb[:, hh * DH:(hh + 1) * DH]
                s_parts = []
                for r in range(N_DEV):
                    kk = k_full[r, b, hh]
                    s = lax.dot_general(
                        q, kk, (((1,), (1,)), ((), ())),
                        preferred_element_type=jnp.float32) * SCALE
                    s_parts.append(jnp.where(masks[r], s, NEG))
                sc = jnp.concatenate(s_parts, axis=1)
                mmax = jnp.max(sc, axis=1, keepdims=True)
                w = jnp.exp(sc - mmax)
                w = w / jnp.sum(w, axis=1, keepdims=True)
                ctx = jnp.zeros((S_loc, DH), jnp.float32)
                for r in range(N_DEV):
                    ctx = ctx + jnp.dot(
                        w[:, r * S_loc:(r + 1) * S_loc], v_full[r, b, hh],
                        preferred_element_type=jnp.float32)
                ctx_cols.append(ctx)
            ctx_b = jnp.concatenate(ctx_cols, axis=1)
            out_ref[b] = jnp.dot(ctx_b, wo_ref[...],
                                 preferred_element_type=jnp.float32)

    return pl.pallas_call(
        body,
        out_shape=jax.ShapeDtypeStruct((B, S_loc, E), jnp.float32),
        in_specs=[pl.BlockSpec(memory_space=pltpu.VMEM)] * 5,
        out_specs=pl.BlockSpec(memory_space=pltpu.VMEM),
        scratch_shapes=[
            pltpu.VMEM((N_DEV, B, HQ, S_loc, DH), jnp.float32),
            pltpu.VMEM((N_DEV, B, HQ, S_loc, DH), jnp.float32),
            pltpu.SemaphoreType.DMA((N_DEV - 1,)),
            pltpu.SemaphoreType.DMA((N_DEV - 1,)),
            pltpu.SemaphoreType.DMA((N_DEV - 1,)),
            pltpu.SemaphoreType.DMA((N_DEV - 1,)),
        ],
        compiler_params=pltpu.CompilerParams(
            collective_id=0, vmem_limit_bytes=100 * 1024 * 1024),
    )(x, Wq, k_t, v_t, Wo)


# device time: 74791 ns/iter; 4.5381x vs baseline; 4.5381x over previous
import functools

import jax
import jax.numpy as jnp
from jax import lax
from jax.experimental import pallas as pl
from jax.experimental.pallas import tpu as pltpu

N_DEV = 4
WINDOW = 128
NGLOB = 32
HALO = 128
SCALE = 0.125


def kernel(x, Wq, K_ext, V_ext, Wo):
    B, S_loc, E = x.shape
    _, _, HQ, DH = K_ext.shape
    DM = HQ * DH

    k_t = jnp.transpose(K_ext, (0, 2, 1, 3))
    v_t = jnp.transpose(V_ext, (0, 2, 1, 3))

    def body(x_ref, wq_ref, k_ref, v_ref, wo_ref, out_ref,
             klh, krh, vlh, vrh, kg, vg, xg, psend, pbuf,
             hs, hr, xs, xr, gks, gvs, gkr, gvr, pss, pr):
        my = lax.axis_index("i")
        left = lax.rem(my + N_DEV - 1, N_DEV)
        right = lax.rem(my + 1, N_DEV)
        opp = lax.rem(my + 2, N_DEV)

        barrier_sem = pltpu.get_barrier_semaphore()
        for nbr in (left, right, opp):
            pl.semaphore_signal(barrier_sem, inc=1, device_id=(nbr,),
                                device_id_type=pl.DeviceIdType.MESH)
        pl.semaphore_wait(barrier_sem, N_DEV - 1)

        halo_rdmas = []
        for i, (src, dst, tgt) in enumerate((
                (k_ref.at[:, :, pl.ds(0, HALO)], krh, left),
                (k_ref.at[:, :, pl.ds(S_loc - HALO, HALO)], klh, right),
                (v_ref.at[:, :, pl.ds(0, HALO)], vrh, left),
                (v_ref.at[:, :, pl.ds(S_loc - HALO, HALO)], vlh, right))):
            r = pltpu.make_async_remote_copy(
                src_ref=src, dst_ref=dst, send_sem=hs.at[i], recv_sem=hr.at[i],
                device_id=(tgt,), device_id_type=pl.DeviceIdType.MESH)
            r.start()
            halo_rdmas.append(r)

        @pl.when(my == 0)
        def _():
            kg[...] = jnp.zeros_like(kg)
            vg[...] = jnp.zeros_like(vg)
            xg[...] = x_ref[:, pl.ds(0, NGLOB)]
            for t in range(1, N_DEV):
                pltpu.make_async_remote_copy(
                    src_ref=xg, dst_ref=xg, send_sem=xs.at[t - 1],
                    recv_sem=xr, device_id=(t,),
                    device_id_type=pl.DeviceIdType.MESH).start()
                pltpu.make_async_remote_copy(
                    src_ref=k_ref.at[:, :, pl.ds(0, NGLOB)], dst_ref=kg,
                    send_sem=gks.at[t - 1], recv_sem=gkr, device_id=(t,),
                    device_id_type=pl.DeviceIdType.MESH).start()
                pltpu.make_async_remote_copy(
                    src_ref=v_ref.at[:, :, pl.ds(0, NGLOB)], dst_ref=vg,
                    send_sem=gvs.at[t - 1], recv_sem=gvr, device_id=(t,),
                    device_id_type=pl.DeviceIdType.MESH).start()

        pbuf[...] = jnp.zeros_like(pbuf)

        Qb = [jnp.dot(x_ref[b], wq_ref[...],
                      preferred_element_type=jnp.float32) for b in range(B)]

        @pl.when(my != 0)
        def _():
            pltpu.make_async_remote_copy(
                src_ref=xg, dst_ref=xg, send_sem=xs.at[0], recv_sem=xr,
                device_id=(0,), device_id_type=pl.DeviceIdType.MESH
            ).wait_recv()

        q32 = [jnp.dot(xg[b], wq_ref[...],
                       preferred_element_type=jnp.float32) for b in range(B)]
        kj_loc = my * S_loc + lax.broadcasted_iota(jnp.int32, (NGLOB, S_loc), 1)
        pmask = kj_loc >= (S_loc + HALO)
        for b in range(B):
            for hh in range(HQ):
                qp = q32[b][:, hh * DH:(hh + 1) * DH]
                sp = lax.dot_general(
                    qp, k_ref[b, hh], (((1,), (1,)), ((), ())),
                    preferred_element_type=jnp.float32) * SCALE
                wp = jnp.where(pmask, jnp.exp(sp), 0.0)
                lp = jnp.sum(wp, axis=1, keepdims=True)
                accp = jnp.dot(wp, v_ref[b, hh],
                               preferred_element_type=jnp.float32)
                psend[b, hh] = jnp.concatenate(
                    [accp, jnp.broadcast_to(lp, (NGLOB, DH))], axis=1)
        for d in range(1, N_DEV):
            @pl.when(my == d)
            def _(d=d):
                pltpu.make_async_remote_copy(
                    src_ref=psend, dst_ref=pbuf.at[d], send_sem=pss,
                    recv_sem=pr.at[d], device_id=(0,),
                    device_id_type=pl.DeviceIdType.MESH).start()

        for r in halo_rdmas:
            r.wait_recv()

        @pl.when(my != 0)
        def _():
            pltpu.make_async_remote_copy(
                src_ref=kg, dst_ref=kg, send_sem=gks.at[0], recv_sem=gkr,
                device_id=(0,), device_id_type=pl.DeviceIdType.MESH
            ).wait_recv()
            pltpu.make_async_remote_copy(
                src_ref=vg, dst_ref=vg, send_sem=gvs.at[0], recv_sem=gvr,
                device_id=(0,), device_id_type=pl.DeviceIdType.MESH
            ).wait_recv()

        @pl.when(my == 0)
        def _():
            for d in range(1, N_DEV):
                pltpu.make_async_remote_copy(
                    src_ref=psend, dst_ref=pbuf.at[d], send_sem=pss,
                    recv_sem=pr.at[d], device_id=(0,),
                    device_id_type=pl.DeviceIdType.MESH).wait_recv()

        def orig_mask(qi, kj):
            return (jnp.abs(qi - kj) <= WINDOW) | (kj < NGLOB) | (qi < NGLOB)

        def qi_col(w):
            return my * S_loc + lax.broadcasted_iota(jnp.int32, (S_loc, w), 0)

        kj_own = my * S_loc + lax.broadcasted_iota(
            jnp.int32, (S_loc, S_loc), 1)
        kj_l = my * S_loc - HALO + lax.broadcasted_iota(
            jnp.int32, (S_loc, HALO), 1)
        kj_r = (my + 1) * S_loc + lax.broadcasted_iota(
            jnp.int32, (S_loc, HALO), 1)
        kj_g = lax.broadcasted_iota(jnp.int32, (S_loc, NGLOB), 1)
        mask = jnp.concatenate([
            orig_mask(qi_col(S_loc), kj_own),
            orig_mask(qi_col(HALO), kj_l) & (my > 0),
            orig_mask(qi_col(HALO), kj_r) & (my < N_DEV - 1),
            orig_mask(qi_col(NGLOB), kj_g) & (my != 0),
        ], axis=1)

        for b in range(B):
            ctx_cols = []
            k_cat = [jnp.concatenate(
                [k_ref[b, hh], klh[b, hh], krh[b, hh], kg[b, hh]], axis=0)
                for hh in range(HQ)]
            v_cat = [jnp.concatenate(
                [v_ref[b, hh], vlh[b, hh], vrh[b, hh], vg[b, hh]], axis=0)
                for hh in range(HQ)]
            for hh in range(HQ):
                q = Qb[b][:, hh * DH:(hh + 1) * DH]
                sc = lax.dot_general(
                    q, k_cat[hh], (((1,), (1,)), ((), ())),
                    preferred_element_type=jnp.float32) * SCALE
                w = jnp.where(mask, jnp.exp(sc), 0.0)
                l = jnp.sum(w, axis=1, keepdims=True)
                acc = jnp.dot(w, v_cat[hh],
                              preferred_element_type=jnp.float32)
                p_acc = sum(pbuf[d, b, hh, :, 0:DH] for d in range(N_DEV))
                p_l = sum(pbuf[d, b, hh, :, DH:DH + 1] for d in range(N_DEV))
                ctx = jnp.concatenate([
                    (acc[0:NGLOB] + p_acc) / (l[0:NGLOB] + p_l),
                    acc[NGLOB:] / l[NGLOB:],
                ], axis=0)
                ctx_cols.append(ctx)
            ctx_b = jnp.concatenate(ctx_cols, axis=1)
            out_ref[b] = jnp.dot(ctx_b, wo_ref[...],
                                 preferred_element_type=jnp.float32)

        for r in halo_rdmas:
            r.wait_send()

        @pl.when(my == 0)
        def _():
            for t in range(1, N_DEV):
                pltpu.make_async_remote_copy(
                    src_ref=xg, dst_ref=xg, send_sem=xs.at[t - 1],
                    recv_sem=xr, device_id=(t,),
                    device_id_type=pl.DeviceIdType.MESH).wait_send()
                pltpu.make_async_remote_copy(
                    src_ref=k_ref.at[:, :, pl.ds(0, NGLOB)], dst_ref=kg,
                    send_sem=gks.at[t - 1], recv_sem=gkr, device_id=(t,),
                    device_id_type=pl.DeviceIdType.MESH).wait_send()
                pltpu.make_async_remote_copy(
                    src_ref=v_ref.at[:, :, pl.ds(0, NGLOB)], dst_ref=vg,
                    send_sem=gvs.at[t - 1], recv_sem=gvr, device_id=(t,),
                    device_id_type=pl.DeviceIdType.MESH).wait_send()

        @pl.when(my != 0)
        def _():
            pltpu.make_async_remote_copy(
                src_ref=psend, dst_ref=pbuf.at[1], send_sem=pss,
                recv_sem=pr.at[1], device_id=(0,),
                device_id_type=pl.DeviceIdType.MESH).wait_send()

        @functools.partial(pl.run_scoped,
                           second_barrier=pltpu.SemaphoreType.REGULAR)
        def _(second_barrier):
            for nbr in (left, right, opp):
                pl.semaphore_signal(second_barrier, inc=1, device_id=(nbr,),
                                    device_id_type=pl.DeviceIdType.MESH)
            pl.semaphore_wait(second_barrier, N_DEV - 1)

    return pl.pallas_call(
        body,
        out_shape=jax.ShapeDtypeStruct((B, S_loc, E), jnp.float32),
        in_specs=[pl.BlockSpec(memory_space=pltpu.VMEM)] * 5,
        out_specs=pl.BlockSpec(memory_space=pltpu.VMEM),
        scratch_shapes=[
            pltpu.VMEM((B, HQ, HALO, DH), jnp.float32),
            pltpu.VMEM((B, HQ, HALO, DH), jnp.float32),
            pltpu.VMEM((B, HQ, HALO, DH), jnp.float32),
            pltpu.VMEM((B, HQ, HALO, DH), jnp.float32),
            pltpu.VMEM((B, HQ, NGLOB, DH), jnp.float32),
            pltpu.VMEM((B, HQ, NGLOB, DH), jnp.float32),
            pltpu.VMEM((B, NGLOB, E), jnp.float32),
            pltpu.VMEM((B, HQ, NGLOB, 2 * DH), jnp.float32),
            pltpu.VMEM((N_DEV, B, HQ, NGLOB, 2 * DH), jnp.float32),
            pltpu.SemaphoreType.DMA((4,)),
            pltpu.SemaphoreType.DMA((4,)),
            pltpu.SemaphoreType.DMA((3,)),
            pltpu.SemaphoreType.DMA,
            pltpu.SemaphoreType.DMA((3,)),
            pltpu.SemaphoreType.DMA((3,)),
            pltpu.SemaphoreType.DMA,
            pltpu.SemaphoreType.DMA,
            pltpu.SemaphoreType.DMA,
            pltpu.SemaphoreType.DMA((N_DEV,)),
        ],
        compiler_params=pltpu.CompilerParams(
            collective_id=0, vmem_limit_bytes=100 * 1024 * 1024),
    )(x, Wq, k_t, v_t, Wo)


# device time: 59358 ns/iter; 5.7180x vs baseline; 1.2600x over previous
import functools

import jax
import jax.numpy as jnp
from jax import lax
from jax.experimental import pallas as pl
from jax.experimental.pallas import tpu as pltpu

N_DEV = 4
WINDOW = 128
NGLOB = 32
HALO = 128
SCALE = 0.125
BF16 = jnp.bfloat16


def kernel(x, Wq, K_ext, V_ext, Wo):
    B, S_loc, E = x.shape
    _, _, HQ, DH = K_ext.shape

    k_t = jnp.transpose(K_ext, (0, 2, 1, 3))
    v_t = jnp.transpose(V_ext, (0, 2, 1, 3))

    def body(x_ref, wq_ref, k_ref, v_ref, wo_ref, out_ref,
             kA, kB, vA, vB, klh, krh, vlh, vrh, kg, vg, xg, psend, pbuf,
             hs, hr, xs, xr, gks, gvs, gkr, gvr, pss, pr):
        my = lax.axis_index("i")
        left = lax.rem(my + N_DEV - 1, N_DEV)
        right = lax.rem(my + 1, N_DEV)
        opp = lax.rem(my + 2, N_DEV)

        barrier_sem = pltpu.get_barrier_semaphore()
        for nbr in (left, right, opp):
            pl.semaphore_signal(barrier_sem, inc=1, device_id=(nbr,),
                                device_id_type=pl.DeviceIdType.MESH)
        pl.semaphore_wait(barrier_sem, N_DEV - 1)

        kA[...] = k_ref[:, :, pl.ds(0, HALO)].astype(BF16)
        kB[...] = k_ref[:, :, pl.ds(S_loc - HALO, HALO)].astype(BF16)
        vA[...] = v_ref[:, :, pl.ds(0, HALO)].astype(BF16)
        vB[...] = v_ref[:, :, pl.ds(S_loc - HALO, HALO)].astype(BF16)

        halo_rdmas = []
        for i, (src, dst, tgt) in enumerate(
                ((kA, krh, left), (kB, klh, right),
                 (vA, vrh, left), (vB, vlh, right))):
            r = pltpu.make_async_remote_copy(
                src_ref=src, dst_ref=dst, send_sem=hs.at[i], recv_sem=hr.at[i],
                device_id=(tgt,), device_id_type=pl.DeviceIdType.MESH)
            r.start()
            halo_rdmas.append(r)

        @pl.when(my == 0)
        def _():
            xg[...] = x_ref[:, pl.ds(0, NGLOB)].astype(BF16)
            for t in range(1, N_DEV):
                pltpu.make_async_remote_copy(
                    src_ref=xg, dst_ref=xg, send_sem=xs.at[t - 1],
                    recv_sem=xr, device_id=(t,),
                    device_id_type=pl.DeviceIdType.MESH).start()
                pltpu.make_async_remote_copy(
                    src_ref=kA.at[:, :, pl.ds(0, NGLOB)], dst_ref=kg,
                    send_sem=gks.at[t - 1], recv_sem=gkr, device_id=(t,),
                    device_id_type=pl.DeviceIdType.MESH).start()
                pltpu.make_async_remote_copy(
                    src_ref=vA.at[:, :, pl.ds(0, NGLOB)], dst_ref=vg,
                    send_sem=gvs.at[t - 1], recv_sem=gvr, device_id=(t,),
                    device_id_type=pl.DeviceIdType.MESH).start()
            kg[...] = jnp.zeros_like(kg)
            vg[...] = jnp.zeros_like(vg)

        pbuf[...] = jnp.zeros_like(pbuf)

        wq16 = wq_ref[...].astype(BF16)
        Qb = [jnp.dot(x_ref[b].astype(BF16), wq16,
                      preferred_element_type=jnp.float32) for b in range(B)]
        k16 = [[k_ref[b, hh].astype(BF16) for hh in range(HQ)]
               for b in range(B)]
        v16 = [[v_ref[b, hh].astype(BF16) for hh in range(HQ)]
               for b in range(B)]

        @pl.when(my != 0)
        def _():
            pltpu.make_async_remote_copy(
                src_ref=xg, dst_ref=xg, send_sem=xs.at[0], recv_sem=xr,
                device_id=(0,), device_id_type=pl.DeviceIdType.MESH
            ).wait_recv()

        q32 = [jnp.dot(xg[b], wq16,
                       preferred_element_type=jnp.float32) for b in range(B)]
        kj_loc = my * S_loc + lax.broadcasted_iota(jnp.int32, (NGLOB, S_loc), 1)
        pmask = kj_loc >= (S_loc + HALO)
        for b in range(B):
            for hh in range(HQ):
                qp = q32[b][:, hh * DH:(hh + 1) * DH].astype(BF16)
                sp = lax.dot_general(
                    qp, k16[b][hh], (((1,), (1,)), ((), ())),
                    preferred_element_type=jnp.float32) * SCALE
                wp = jnp.where(pmask, jnp.exp(sp), 0.0)
                lp = jnp.sum(wp, axis=1, keepdims=True)
                accp = jnp.dot(wp.astype(BF16), v16[b][hh],
                               preferred_element_type=jnp.float32)
                psend[b, hh] = jnp.concatenate(
                    [accp, jnp.broadcast_to(lp, (NGLOB, DH))], axis=1)
        for d in range(1, N_DEV):
            @pl.when(my == d)
            def _(d=d):
                pltpu.make_async_remote_copy(
                    src_ref=psend, dst_ref=pbuf.at[d], send_sem=pss,
                    recv_sem=pr.at[d], device_id=(0,),
                    device_id_type=pl.DeviceIdType.MESH).start()

        def orig_mask(qi, kj):
            return (jnp.abs(qi - kj) <= WINDOW) | (kj < NGLOB) | (qi < NGLOB)

        def qi_col(w):
            return my * S_loc + lax.broadcasted_iota(jnp.int32, (S_loc, w), 0)

        kj_own = my * S_loc + lax.broadcasted_iota(
            jnp.int32, (S_loc, S_loc), 1)
        mask_own = orig_mask(qi_col(S_loc), kj_own)
        kj_l = my * S_loc - HALO + lax.broadcasted_iota(
            jnp.int32, (S_loc, HALO), 1)
        kj_r = (my + 1) * S_loc + lax.broadcasted_iota(
            jnp.int32, (S_loc, HALO), 1)
        kj_g = lax.broadcasted_iota(jnp.int32, (S_loc, NGLOB), 1)
        mask_h = jnp.concatenate([
            orig_mask(qi_col(HALO), kj_l) & (my > 0),
            orig_mask(qi_col(HALO), kj_r) & (my < N_DEV - 1),
            orig_mask(qi_col(NGLOB), kj_g) & (my != 0),
        ], axis=1)

        q16 = [[Qb[b][:, hh * DH:(hh + 1) * DH].astype(BF16)
                for hh in range(HQ)] for b in range(B)]
        accA = [[None] * HQ for _ in range(B)]
        lA = [[None] * HQ for _ in range(B)]
        for b in range(B):
            for hh in range(HQ):
                sc = lax.dot_general(
                    q16[b][hh], k16[b][hh], (((1,), (1,)), ((), ())),
                    preferred_element_type=jnp.float32) * SCALE
                w = jnp.where(mask_own, jnp.exp(sc), 0.0)
                lA[b][hh] = jnp.sum(w, axis=1, keepdims=True)
                accA[b][hh] = jnp.dot(w.astype(BF16), v16[b][hh],
                                      preferred_element_type=jnp.float32)

        for r in halo_rdmas:
            r.wait_recv()

        @pl.when(my != 0)
        def _():
            pltpu.make_async_remote_copy(
                src_ref=kg, dst_ref=kg, send_sem=gks.at[0], recv_sem=gkr,
                device_id=(0,), device_id_type=pl.DeviceIdType.MESH
            ).wait_recv()
            pltpu.make_async_remote_copy(
                src_ref=vg, dst_ref=vg, send_sem=gvs.at[0], recv_sem=gvr,
                device_id=(0,), device_id_type=pl.DeviceIdType.MESH
            ).wait_recv()

        @pl.when(my == 0)
        def _():
            for d in range(1, N_DEV):
                pltpu.make_async_remote_copy(
                    src_ref=psend, dst_ref=pbuf.at[d], send_sem=pss,
                    recv_sem=pr.at[d], device_id=(0,),
                    device_id_type=pl.DeviceIdType.MESH).wait_recv()

        wo16 = wo_ref[...].astype(BF16)
        for b in range(B):
            ctx_cols = []
            for hh in range(HQ):
                k_hcat = jnp.concatenate(
                    [klh[b, hh], krh[b, hh], kg[b, hh]], axis=0)
                v_hcat = jnp.concatenate(
                    [vlh[b, hh], vrh[b, hh], vg[b, hh]], axis=0)
                sc = lax.dot_general(
                    q16[b][hh], k_hcat, (((1,), (1,)), ((), ())),
                    preferred_element_type=jnp.float32) * SCALE
                w = jnp.where(mask_h, jnp.exp(sc), 0.0)
                l = lA[b][hh] + jnp.sum(w, axis=1, keepdims=True)
                acc = accA[b][hh] + jnp.dot(
                    w.astype(BF16), v_hcat,
                    preferred_element_type=jnp.float32)
                p_acc = sum(pbuf[d, b, hh, :, 0:DH] for d in range(N_DEV))
                p_l = sum(pbuf[d, b, hh, :, DH:DH + 1] for d in range(N_DEV))
                ctx = jnp.concatenate([
                    (acc[0:NGLOB] + p_acc) / (l[0:NGLOB] + p_l),
                    acc[NGLOB:] / l[NGLOB:],
                ], axis=0)
                ctx_cols.append(ctx)
            ctx_b = jnp.concatenate(ctx_cols, axis=1)
            out_ref[b] = jnp.dot(ctx_b.astype(BF16), wo16,
                                 preferred_element_type=jnp.float32)

        for r in halo_rdmas:
            r.wait_send()

        @pl.when(my == 0)
        def _():
            for t in range(1, N_DEV):
                pltpu.make_async_remote_copy(
                    src_ref=xg, dst_ref=xg, send_sem=xs.at[t - 1],
                    recv_sem=xr, device_id=(t,),
                    device_id_type=pl.DeviceIdType.MESH).wait_send()
                pltpu.make_async_remote_copy(
                    src_ref=kA.at[:, :, pl.ds(0, NGLOB)], dst_ref=kg,
                    send_sem=gks.at[t - 1], recv_sem=gkr, device_id=(t,),
                    device_id_type=pl.DeviceIdType.MESH).wait_send()
                pltpu.make_async_remote_copy(
                    src_ref=vA.at[:, :, pl.ds(0, NGLOB)], dst_ref=vg,
                    send_sem=gvs.at[t - 1], recv_sem=gvr, device_id=(t,),
                    device_id_type=pl.DeviceIdType.MESH).wait_send()

        @pl.when(my != 0)
        def _():
            pltpu.make_async_remote_copy(
                src_ref=psend, dst_ref=pbuf.at[1], send_sem=pss,
                recv_sem=pr.at[1], device_id=(0,),
                device_id_type=pl.DeviceIdType.MESH).wait_send()

        @functools.partial(pl.run_scoped,
                           second_barrier=pltpu.SemaphoreType.REGULAR)
        def _(second_barrier):
            for nbr in (left, right, opp):
                pl.semaphore_signal(second_barrier, inc=1, device_id=(nbr,),
                                    device_id_type=pl.DeviceIdType.MESH)
            pl.semaphore_wait(second_barrier, N_DEV - 1)

    return pl.pallas_call(
        body,
        out_shape=jax.ShapeDtypeStruct((B, S_loc, E), jnp.float32),
        in_specs=[pl.BlockSpec(memory_space=pltpu.VMEM)] * 5,
        out_specs=pl.BlockSpec(memory_space=pltpu.VMEM),
        scratch_shapes=[
            pltpu.VMEM((B, HQ, HALO, DH), BF16),
            pltpu.VMEM((B, HQ, HALO, DH), BF16),
            pltpu.VMEM((B, HQ, HALO, DH), BF16),
            pltpu.VMEM((B, HQ, HALO, DH), BF16),
            pltpu.VMEM((B, HQ, HALO, DH), BF16),
            pltpu.VMEM((B, HQ, HALO, DH), BF16),
            pltpu.VMEM((B, HQ, HALO, DH), BF16),
            pltpu.VMEM((B, HQ, HALO, DH), BF16),
            pltpu.VMEM((B, HQ, NGLOB, DH), BF16),
            pltpu.VMEM((B, HQ, NGLOB, DH), BF16),
            pltpu.VMEM((B, NGLOB, E), BF16),
            pltpu.VMEM((B, HQ, NGLOB, 2 * DH), jnp.float32),
            pltpu.VMEM((N_DEV, B, HQ, NGLOB, 2 * DH), jnp.float32),
            pltpu.SemaphoreType.DMA((4,)),
            pltpu.SemaphoreType.DMA((4,)),
            pltpu.SemaphoreType.DMA((3,)),
            pltpu.SemaphoreType.DMA,
            pltpu.SemaphoreType.DMA((3,)),
            pltpu.SemaphoreType.DMA((3,)),
            pltpu.SemaphoreType.DMA,
            pltpu.SemaphoreType.DMA,
            pltpu.SemaphoreType.DMA,
            pltpu.SemaphoreType.DMA((N_DEV,)),
        ],
        compiler_params=pltpu.CompilerParams(
            collective_id=0, vmem_limit_bytes=100 * 1024 * 1024),
    )(x, Wq, k_t, v_t, Wo)


# device time: 48517 ns/iter; 6.9956x vs baseline; 1.2234x over previous
import functools

import jax
import jax.numpy as jnp
from jax import lax
from jax.experimental import pallas as pl
from jax.experimental.pallas import tpu as pltpu

N_DEV = 4
WINDOW = 128
NGLOB = 32
HALO = 128
SCALE = 0.125
BF16 = jnp.bfloat16


def kernel(x, Wq, K_ext, V_ext, Wo):
    B, S_loc, E = x.shape
    _, _, HQ, DH = K_ext.shape
    DM = HQ * DH

    k_r = K_ext.reshape(B, S_loc, DM)
    v_r = V_ext.reshape(B, S_loc, DM)

    def body(x_ref, wq_ref, k_ref, v_ref, wo_ref, out_ref,
             kA, kB, vA, vB, klh, krh, vlh, vrh, kg, vg, xg, psend, pbuf,
             hs, hr, xs, xr, gks, gvs, gkr, gvr, pss, pr):
        my = lax.axis_index("i")
        left = lax.rem(my + N_DEV - 1, N_DEV)
        right = lax.rem(my + 1, N_DEV)
        opp = lax.rem(my + 2, N_DEV)

        barrier_sem = pltpu.get_barrier_semaphore()
        for nbr in (left, right, opp):
            pl.semaphore_signal(barrier_sem, inc=1, device_id=(nbr,),
                                device_id_type=pl.DeviceIdType.MESH)
        pl.semaphore_wait(barrier_sem, N_DEV - 1)

        kA[...] = k_ref[:, pl.ds(0, HALO)].astype(BF16)
        kB[...] = k_ref[:, pl.ds(S_loc - HALO, HALO)].astype(BF16)
        vA[...] = v_ref[:, pl.ds(0, HALO)].astype(BF16)
        vB[...] = v_ref[:, pl.ds(S_loc - HALO, HALO)].astype(BF16)

        halo_rdmas = []
        for i, (src, dst, tgt) in enumerate(
                ((kA, krh, left), (kB, klh, right),
                 (vA, vrh, left), (vB, vlh, right))):
            r = pltpu.make_async_remote_copy(
                src_ref=src, dst_ref=dst, send_sem=hs.at[i], recv_sem=hr.at[i],
                device_id=(tgt,), device_id_type=pl.DeviceIdType.MESH)
            r.start()
            halo_rdmas.append(r)

        @pl.when(my == 0)
        def _():
            xg[...] = x_ref[:, pl.ds(0, NGLOB)].astype(BF16)
            for t in range(1, N_DEV):
                pltpu.make_async_remote_copy(
                    src_ref=xg, dst_ref=xg, send_sem=xs.at[t - 1],
                    recv_sem=xr, device_id=(t,),
                    device_id_type=pl.DeviceIdType.MESH).start()
                pltpu.make_async_remote_copy(
                    src_ref=kA.at[:, pl.ds(0, NGLOB)], dst_ref=kg,
                    send_sem=gks.at[t - 1], recv_sem=gkr, device_id=(t,),
                    device_id_type=pl.DeviceIdType.MESH).start()
                pltpu.make_async_remote_copy(
                    src_ref=vA.at[:, pl.ds(0, NGLOB)], dst_ref=vg,
                    send_sem=gvs.at[t - 1], recv_sem=gvr, device_id=(t,),
                    device_id_type=pl.DeviceIdType.MESH).start()
            kg[...] = jnp.zeros_like(kg)
            vg[...] = jnp.zeros_like(vg)

        pbuf[...] = jnp.zeros_like(pbuf)

        wq16 = wq_ref[...].astype(BF16)
        Qb = [jnp.dot(x_ref[b].astype(BF16), wq16,
                      preferred_element_type=jnp.float32) for b in range(B)]
        k16 = [k_ref[b].astype(BF16) for b in range(B)]
        v16 = [v_ref[b].astype(BF16) for b in range(B)]
        ones_own = jnp.ones((S_loc, 1), BF16)

        def hsl(hh):
            return slice(hh * DH, (hh + 1) * DH)

        @pl.when(my != 0)
        def _():
            pltpu.make_async_remote_copy(
                src_ref=xg, dst_ref=xg, send_sem=xs.at[0], recv_sem=xr,
                device_id=(0,), device_id_type=pl.DeviceIdType.MESH
            ).wait_recv()
            q32 = [jnp.dot(xg[b], wq16, preferred_element_type=jnp.float32)
                   for b in range(B)]
            kj_loc = my * S_loc + lax.broadcasted_iota(
                jnp.int32, (NGLOB, S_loc), 1)
            pmask = kj_loc >= (S_loc + HALO)
            for b in range(B):
                for hh in range(HQ):
                    qp = q32[b][:, hsl(hh)].astype(BF16)
                    sp = lax.dot_general(
                        qp, k16[b][:, hsl(hh)], (((1,), (1,)), ((), ())),
                        preferred_element_type=jnp.float32) * SCALE
                    wp = jnp.where(pmask, jnp.exp(sp), 0.0)
                    lp = jnp.sum(wp, axis=1, keepdims=True)
                    accp = jnp.dot(wp.astype(BF16), v16[b][:, hsl(hh)],
                                   preferred_element_type=jnp.float32)
                    psend[b, hh] = jnp.concatenate(
                        [accp, jnp.broadcast_to(lp, (NGLOB, DH))], axis=1)

        for d in range(1, N_DEV):
            @pl.when(my == d)
            def _(d=d):
                pltpu.make_async_remote_copy(
                    src_ref=psend, dst_ref=pbuf.at[d], send_sem=pss,
                    recv_sem=pr.at[d], device_id=(0,),
                    device_id_type=pl.DeviceIdType.MESH).start()

        def orig_mask(qi, kj):
            return (jnp.abs(qi - kj) <= WINDOW) | (kj < NGLOB) | (qi < NGLOB)

        def qi_col(w):
            return my * S_loc + lax.broadcasted_iota(jnp.int32, (S_loc, w), 0)

        kj_own = my * S_loc + lax.broadcasted_iota(
            jnp.int32, (S_loc, S_loc), 1)
        mask_own = orig_mask(qi_col(S_loc), kj_own)
        kj_l = my * S_loc - HALO + lax.broadcasted_iota(
            jnp.int32, (S_loc, HALO), 1)
        kj_r = (my + 1) * S_loc + lax.broadcasted_iota(
            jnp.int32, (S_loc, HALO), 1)
        kj_g = lax.broadcasted_iota(jnp.int32, (S_loc, NGLOB), 1)
        mask_h = jnp.concatenate([
            orig_mask(qi_col(HALO), kj_l) & (my > 0),
            orig_mask(qi_col(HALO), kj_r) & (my < N_DEV - 1),
            orig_mask(qi_col(NGLOB), kj_g) & (my != 0),
        ], axis=1)

        q16 = [[Qb[b][:, hsl(hh)].astype(BF16)
                for hh in range(HQ)] for b in range(B)]
        accA = [[None] * HQ for _ in range(B)]
        lA = [[None] * HQ for _ in range(B)]
        for b in range(B):
            for hh in range(HQ):
                sc = lax.dot_general(
                    q16[b][hh], k16[b][:, hsl(hh)], (((1,), (1,)), ((), ())),
                    preferred_element_type=jnp.float32) * SCALE
                w = jnp.where(mask_own, jnp.exp(sc), 0.0).astype(BF16)
                v_aug = jnp.concatenate(
                    [v16[b][:, hsl(hh)], ones_own], axis=1)
                aug = jnp.dot(w, v_aug,
                              preferred_element_type=jnp.float32)
                accA[b][hh] = aug[:, 0:DH]
                lA[b][hh] = aug[:, DH:DH + 1]

        for r in halo_rdmas:
            r.wait_recv()

        @pl.when(my != 0)
        def _():
            pltpu.make_async_remote_copy(
                src_ref=kg, dst_ref=kg, send_sem=gks.at[0], recv_sem=gkr,
                device_id=(0,), device_id_type=pl.DeviceIdType.MESH
            ).wait_recv()
            pltpu.make_async_remote_copy(
                src_ref=vg, dst_ref=vg, send_sem=gvs.at[0], recv_sem=gvr,
                device_id=(0,), device_id_type=pl.DeviceIdType.MESH
            ).wait_recv()

        @pl.when(my == 0)
        def _():
            for d in range(1, N_DEV):
                pltpu.make_async_remote_copy(
                    src_ref=psend, dst_ref=pbuf.at[d], send_sem=pss,
                    recv_sem=pr.at[d], device_id=(0,),
                    device_id_type=pl.DeviceIdType.MESH).wait_recv()

        wo16 = wo_ref[...].astype(BF16)
        ones_h = jnp.ones((2 * HALO + NGLOB, 1), BF16)
        for b in range(B):
            klh_b, krh_b, kg_b = klh[b], krh[b], kg[b]
            vlh_b, vrh_b, vg_b = vlh[b], vrh[b], vg[b]
            ctx_cols = []
            for hh in range(HQ):
                k_hcat = jnp.concatenate(
                    [klh_b[:, hsl(hh)], krh_b[:, hsl(hh)], kg_b[:, hsl(hh)]],
                    axis=0)
                v_hcat = jnp.concatenate(
                    [vlh_b[:, hsl(hh)], vrh_b[:, hsl(hh)], vg_b[:, hsl(hh)]],
                    axis=0)
                v_haug = jnp.concatenate([v_hcat, ones_h], axis=1)
                sc = lax.dot_general(
                    q16[b][hh], k_hcat, (((1,), (1,)), ((), ())),
                    preferred_element_type=jnp.float32) * SCALE
                w = jnp.where(mask_h, jnp.exp(sc), 0.0).astype(BF16)
                aug = jnp.dot(w, v_haug,
                              preferred_element_type=jnp.float32)
                l = lA[b][hh] + aug[:, DH:DH + 1]
                acc = accA[b][hh] + aug[:, 0:DH]
                p_acc = sum(pbuf[d, b, hh, :, 0:DH] for d in range(N_DEV))
                p_l = sum(pbuf[d, b, hh, :, DH:DH + 1] for d in range(N_DEV))
                ctx = jnp.concatenate([
                    (acc[0:NGLOB] + p_acc) / (l[0:NGLOB] + p_l),
                    acc[NGLOB:] / l[NGLOB:],
                ], axis=0)
                ctx_cols.append(ctx)
            ctx_b = jnp.concatenate(ctx_cols, axis=1)
            out_ref[b] = jnp.dot(ctx_b.astype(BF16), wo16,
                                 preferred_element_type=jnp.float32)

        for r in halo_rdmas:
            r.wait_send()

        @pl.when(my == 0)
        def _():
            for t in range(1, N_DEV):
                pltpu.make_async_remote_copy(
                    src_ref=xg, dst_ref=xg, send_sem=xs.at[t - 1],
                    recv_sem=xr, device_id=(t,),
                    device_id_type=pl.DeviceIdType.MESH).wait_send()
                pltpu.make_async_remote_copy(
                    src_ref=kA.at[:, pl.ds(0, NGLOB)], dst_ref=kg,
                    send_sem=gks.at[t - 1], recv_sem=gkr, device_id=(t,),
                    device_id_type=pl.DeviceIdType.MESH).wait_send()
                pltpu.make_async_remote_copy(
                    src_ref=vA.at[:, pl.ds(0, NGLOB)], dst_ref=vg,
                    send_sem=gvs.at[t - 1], recv_sem=gvr, device_id=(t,),
                    device_id_type=pl.DeviceIdType.MESH).wait_send()

        @pl.when(my != 0)
        def _():
            pltpu.make_async_remote_copy(
                src_ref=psend, dst_ref=pbuf.at[1], send_sem=pss,
                recv_sem=pr.at[1], device_id=(0,),
                device_id_type=pl.DeviceIdType.MESH).wait_send()

        @functools.partial(pl.run_scoped,
                           second_barrier=pltpu.SemaphoreType.REGULAR)
        def _(second_barrier):
            for nbr in (left, right, opp):
                pl.semaphore_signal(second_barrier, inc=1, device_id=(nbr,),
                                    device_id_type=pl.DeviceIdType.MESH)
            pl.semaphore_wait(second_barrier, N_DEV - 1)

    return pl.pallas_call(
        body,
        out_shape=jax.ShapeDtypeStruct((B, S_loc, E), jnp.float32),
        in_specs=[pl.BlockSpec(memory_space=pltpu.VMEM)] * 5,
        out_specs=pl.BlockSpec(memory_space=pltpu.VMEM),
        scratch_shapes=[
            pltpu.VMEM((B, HALO, DM), BF16),
            pltpu.VMEM((B, HALO, DM), BF16),
            pltpu.VMEM((B, HALO, DM), BF16),
            pltpu.VMEM((B, HALO, DM), BF16),
            pltpu.VMEM((B, HALO, DM), BF16),
            pltpu.VMEM((B, HALO, DM), BF16),
            pltpu.VMEM((B, HALO, DM), BF16),
            pltpu.VMEM((B, HALO, DM), BF16),
            pltpu.VMEM((B, NGLOB, DM), BF16),
            pltpu.VMEM((B, NGLOB, DM), BF16),
            pltpu.VMEM((B, NGLOB, E), BF16),
            pltpu.VMEM((B, HQ, NGLOB, 2 * DH), jnp.float32),
            pltpu.VMEM((N_DEV, B, HQ, NGLOB, 2 * DH), jnp.float32),
            pltpu.SemaphoreType.DMA((4,)),
            pltpu.SemaphoreType.DMA((4,)),
            pltpu.SemaphoreType.DMA((3,)),
            pltpu.SemaphoreType.DMA,
            pltpu.SemaphoreType.DMA((3,)),
            pltpu.SemaphoreType.DMA((3,)),
            pltpu.SemaphoreType.DMA,
            pltpu.SemaphoreType.DMA,
            pltpu.SemaphoreType.DMA,
            pltpu.SemaphoreType.DMA((N_DEV,)),
        ],
        compiler_params=pltpu.CompilerParams(
            collective_id=0, vmem_limit_bytes=100 * 1024 * 1024),
    )(x, Wq, k_r, v_r, Wo)


# device time: 42738 ns/iter; 7.9416x vs baseline; 1.1352x over previous
import functools

import jax
import jax.numpy as jnp
from jax import lax
from jax.experimental import pallas as pl
from jax.experimental.pallas import tpu as pltpu

N_DEV = 4
WINDOW = 128
NGLOB = 32
HALO = 128
SCALE = 0.125
BF16 = jnp.bfloat16


def kernel(x, Wq, K_ext, V_ext, Wo):
    B, S_loc, E = x.shape
    _, _, HQ, DH = K_ext.shape
    DM = HQ * DH

    k_r = K_ext.reshape(B, S_loc, DM)
    v_r = V_ext.reshape(B, S_loc, DM)

    def body(x_ref, wq_ref, k_ref, v_ref, wo_ref, out_ref,
             kA, kB, vA, vB, klh, krh, vlh, vrh, kg, vg, xg, psend, pbuf,
             hs, hr, xs, xr, gks, gvs, gkr, gvr, pss, pr):
        my = lax.axis_index("i")
        left = lax.rem(my + N_DEV - 1, N_DEV)
        right = lax.rem(my + 1, N_DEV)
        opp = lax.rem(my + 2, N_DEV)

        barrier_sem = pltpu.get_barrier_semaphore()
        for nbr in (left, right, opp):
            pl.semaphore_signal(barrier_sem, inc=1, device_id=(nbr,),
                                device_id_type=pl.DeviceIdType.MESH)
        pl.semaphore_wait(barrier_sem, N_DEV - 1)

        @pl.when(my == 0)
        def _():
            xg[...] = x_ref[:, pl.ds(0, NGLOB)].astype(BF16)
            for t in range(1, N_DEV):
                pltpu.make_async_remote_copy(
                    src_ref=xg, dst_ref=xg, send_sem=xs.at[t - 1],
                    recv_sem=xr, device_id=(t,),
                    device_id_type=pl.DeviceIdType.MESH).start()

        kA[...] = k_ref[:, pl.ds(0, HALO)].astype(BF16)
        kB[...] = k_ref[:, pl.ds(S_loc - HALO, HALO)].astype(BF16)
        vA[...] = v_ref[:, pl.ds(0, HALO)].astype(BF16)
        vB[...] = v_ref[:, pl.ds(S_loc - HALO, HALO)].astype(BF16)

        halo_rdmas = []
        for i, (src, dst, tgt) in enumerate(
                ((kA, krh, left), (kB, klh, right),
                 (vA, vrh, left), (vB, vlh, right))):
            r = pltpu.make_async_remote_copy(
                src_ref=src, dst_ref=dst, send_sem=hs.at[i], recv_sem=hr.at[i],
                device_id=(tgt,), device_id_type=pl.DeviceIdType.MESH)
            r.start()
            halo_rdmas.append(r)

        @pl.when(my == 0)
        def _():
            for t in range(1, N_DEV):
                pltpu.make_async_remote_copy(
                    src_ref=kA.at[:, pl.ds(0, NGLOB)], dst_ref=kg,
                    send_sem=gks.at[t - 1], recv_sem=gkr, device_id=(t,),
                    device_id_type=pl.DeviceIdType.MESH).start()
                pltpu.make_async_remote_copy(
                    src_ref=vA.at[:, pl.ds(0, NGLOB)], dst_ref=vg,
                    send_sem=gvs.at[t - 1], recv_sem=gvr, device_id=(t,),
                    device_id_type=pl.DeviceIdType.MESH).start()
            kg[...] = jnp.zeros_like(kg)
            vg[...] = jnp.zeros_like(vg)

        pbuf[...] = jnp.zeros_like(pbuf)

        wq16 = wq_ref[...].astype(BF16)
        Qb = [jnp.dot(x_ref[b].astype(BF16), wq16,
                      preferred_element_type=jnp.float32) for b in range(B)]
        k16 = [k_ref[b].astype(BF16) for b in range(B)]
        v16 = [v_ref[b].astype(BF16) for b in range(B)]
        ones_own = jnp.ones((S_loc, 1), BF16)

        def hsl(hh):
            return slice(hh * DH, (hh + 1) * DH)

        @pl.when(my != 0)
        def _():
            pltpu.make_async_remote_copy(
                src_ref=xg, dst_ref=xg, send_sem=xs.at[0], recv_sem=xr,
                device_id=(0,), device_id_type=pl.DeviceIdType.MESH
            ).wait_recv()
            q32 = [jnp.dot(xg[b], wq16, preferred_element_type=jnp.float32)
                   for b in range(B)]
            kj_loc = my * S_loc + lax.broadcasted_iota(
                jnp.int32, (NGLOB, S_loc), 1)
            pmask = kj_loc >= (S_loc + HALO)
            for b in range(B):
                for hh in range(HQ):
                    qp = q32[b][:, hsl(hh)].astype(BF16)
                    sp = lax.dot_general(
                        qp, k16[b][:, hsl(hh)], (((1,), (1,)), ((), ())),
                        preferred_element_type=jnp.float32) * SCALE
                    wp = jnp.where(pmask, jnp.exp(sp), 0.0)
                    lp = jnp.sum(wp, axis=1, keepdims=True)
                    accp = jnp.dot(wp.astype(BF16), v16[b][:, hsl(hh)],
                                   preferred_element_type=jnp.float32)
                    psend[b, hh] = jnp.concatenate(
                        [accp, jnp.broadcast_to(lp, (NGLOB, DH))], axis=1)

        for d in range(1, N_DEV):
            @pl.when(my == d)
            def _(d=d):
                pltpu.make_async_remote_copy(
                    src_ref=psend, dst_ref=pbuf.at[d], send_sem=pss,
                    recv_sem=pr.at[d], device_id=(0,),
                    device_id_type=pl.DeviceIdType.MESH).start()

        def orig_mask(qi, kj):
            return (jnp.abs(qi - kj) <= WINDOW) | (kj < NGLOB) | (qi < NGLOB)

        def qi_col(w):
            return my * S_loc + lax.broadcasted_iota(jnp.int32, (S_loc, w), 0)

        kj_own = my * S_loc + lax.broadcasted_iota(
            jnp.int32, (S_loc, S_loc), 1)
        mask_own = orig_mask(qi_col(S_loc), kj_own)
        kj_l = my * S_loc - HALO + lax.broadcasted_iota(
            jnp.int32, (S_loc, HALO), 1)
        kj_r = (my + 1) * S_loc + lax.broadcasted_iota(
            jnp.int32, (S_loc, HALO), 1)
        kj_g = lax.broadcasted_iota(jnp.int32, (S_loc, NGLOB), 1)
        mask_h = jnp.concatenate([
            orig_mask(qi_col(HALO), kj_l) & (my > 0),
            orig_mask(qi_col(HALO), kj_r) & (my < N_DEV - 1),
            orig_mask(qi_col(NGLOB), kj_g) & (my != 0),
        ], axis=1)

        q16 = [[Qb[b][:, hsl(hh)].astype(BF16)
                for hh in range(HQ)] for b in range(B)]
        accA = [[None] * HQ for _ in range(B)]
        lA = [[None] * HQ for _ in range(B)]
        for b in range(B):
            for hh in range(HQ):
                sc = lax.dot_general(
                    q16[b][hh], k16[b][:, hsl(hh)], (((1,), (1,)), ((), ())),
                    preferred_element_type=jnp.float32) * SCALE
                w = jnp.where(mask_own, jnp.exp(sc), 0.0).astype(BF16)
                v_aug = jnp.concatenate(
                    [v16[b][:, hsl(hh)], ones_own], axis=1)
                aug = jnp.dot(w, v_aug,
                              preferred_element_type=jnp.float32)
                accA[b][hh] = aug[:, 0:DH]
                lA[b][hh] = aug[:, DH:DH + 1]

        for r in halo_rdmas:
            r.wait_recv()

        @pl.when(my != 0)
        def _():
            pltpu.make_async_remote_copy(
                src_ref=kg, dst_ref=kg, send_sem=gks.at[0], recv_sem=gkr,
                device_id=(0,), device_id_type=pl.DeviceIdType.MESH
            ).wait_recv()
            pltpu.make_async_remote_copy(
                src_ref=vg, dst_ref=vg, send_sem=gvs.at[0], recv_sem=gvr,
                device_id=(0,), device_id_type=pl.DeviceIdType.MESH
            ).wait_recv()

        @pl.when(my == 0)
        def _():
            for d in range(1, N_DEV):
                pltpu.make_async_remote_copy(
                    src_ref=psend, dst_ref=pbuf.at[d], send_sem=pss,
                    recv_sem=pr.at[d], device_id=(0,),
                    device_id_type=pl.DeviceIdType.MESH).wait_recv()

        wo16 = wo_ref[...].astype(BF16)
        ones_h = jnp.ones((2 * HALO + NGLOB, 1), BF16)
        for b in range(B):
            klh_b, krh_b, kg_b = klh[b], krh[b], kg[b]
            vlh_b, vrh_b, vg_b = vlh[b], vrh[b], vg[b]
            ctx_cols = []
            for hh in range(HQ):
                k_hcat = jnp.concatenate(
                    [klh_b[:, hsl(hh)], krh_b[:, hsl(hh)], kg_b[:, hsl(hh)]],
                    axis=0)
                v_hcat = jnp.concatenate(
                    [vlh_b[:, hsl(hh)], vrh_b[:, hsl(hh)], vg_b[:, hsl(hh)]],
                    axis=0)
                v_haug = jnp.concatenate([v_hcat, ones_h], axis=1)
                sc = lax.dot_general(
                    q16[b][hh], k_hcat, (((1,), (1,)), ((), ())),
                    preferred_element_type=jnp.float32) * SCALE
                w = jnp.where(mask_h, jnp.exp(sc), 0.0).astype(BF16)
                aug = jnp.dot(w, v_haug,
                              preferred_element_type=jnp.float32)
                l = lA[b][hh] + aug[:, DH:DH + 1]
                acc = accA[b][hh] + aug[:, 0:DH]
                p_acc = sum(pbuf[d, b, hh, :, 0:DH] for d in range(1, N_DEV))
                p_l = sum(pbuf[d, b, hh, :, DH:DH + 1]
                          for d in range(1, N_DEV))
                ctx = jnp.concatenate([
                    (acc[0:NGLOB] + p_acc) / (l[0:NGLOB] + p_l),
                    acc[NGLOB:] / l[NGLOB:],
                ], axis=0)
                ctx_cols.append(ctx)
            ctx_b = jnp.concatenate(ctx_cols, axis=1)
            out_ref[b] = jnp.dot(ctx_b.astype(BF16), wo16,
                                 preferred_element_type=jnp.float32)

        for r in halo_rdmas:
            r.wait_send()

        @pl.when(my == 0)
        def _():
            for t in range(1, N_DEV):
                pltpu.make_async_remote_copy(
                    src_ref=xg, dst_ref=xg, send_sem=xs.at[t - 1],
                    recv_sem=xr, device_id=(t,),
                    device_id_type=pl.DeviceIdType.MESH).wait_send()
                pltpu.make_async_remote_copy(
                    src_ref=kA.at[:, pl.ds(0, NGLOB)], dst_ref=kg,
                    send_sem=gks.at[t - 1], recv_sem=gkr, device_id=(t,),
                    device_id_type=pl.DeviceIdType.MESH).wait_send()
                pltpu.make_async_remote_copy(
                    src_ref=vA.at[:, pl.ds(0, NGLOB)], dst_ref=vg,
                    send_sem=gvs.at[t - 1], recv_sem=gvr, device_id=(t,),
                    device_id_type=pl.DeviceIdType.MESH).wait_send()

        @pl.when(my != 0)
        def _():
            pltpu.make_async_remote_copy(
                src_ref=psend, dst_ref=pbuf.at[1], send_sem=pss,
                recv_sem=pr.at[1], device_id=(0,),
                device_id_type=pl.DeviceIdType.MESH).wait_send()

        @functools.partial(pl.run_scoped,
                           second_barrier=pltpu.SemaphoreType.REGULAR)
        def _(second_barrier):
            for nbr in (left, right, opp):
                pl.semaphore_signal(second_barrier, inc=1, device_id=(nbr,),
                                    device_id_type=pl.DeviceIdType.MESH)
            pl.semaphore_wait(second_barrier, N_DEV - 1)

    return pl.pallas_call(
        body,
        out_shape=jax.ShapeDtypeStruct((B, S_loc, E), jnp.float32),
        in_specs=[pl.BlockSpec(memory_space=pltpu.VMEM)] * 5,
        out_specs=pl.BlockSpec(memory_space=pltpu.VMEM),
        scratch_shapes=[
            pltpu.VMEM((B, HALO, DM), BF16),
            pltpu.VMEM((B, HALO, DM), BF16),
            pltpu.VMEM((B, HALO, DM), BF16),
            pltpu.VMEM((B, HALO, DM), BF16),
            pltpu.VMEM((B, HALO, DM), BF16),
            pltpu.VMEM((B, HALO, DM), BF16),
            pltpu.VMEM((B, HALO, DM), BF16),
            pltpu.VMEM((B, HALO, DM), BF16),
            pltpu.VMEM((B, NGLOB, DM), BF16),
            pltpu.VMEM((B, NGLOB, DM), BF16),
            pltpu.VMEM((B, NGLOB, E), BF16),
            pltpu.VMEM((B, HQ, NGLOB, 2 * DH), jnp.float32),
            pltpu.VMEM((N_DEV, B, HQ, NGLOB, 2 * DH), jnp.float32),
            pltpu.SemaphoreType.DMA((4,)),
            pltpu.SemaphoreType.DMA((4,)),
            pltpu.SemaphoreType.DMA((3,)),
            pltpu.SemaphoreType.DMA,
            pltpu.SemaphoreType.DMA((3,)),
            pltpu.SemaphoreType.DMA((3,)),
            pltpu.SemaphoreType.DMA,
            pltpu.SemaphoreType.DMA,
            pltpu.SemaphoreType.DMA,
            pltpu.SemaphoreType.DMA((N_DEV,)),
        ],
        compiler_params=pltpu.CompilerParams(
            collective_id=0, vmem_limit_bytes=100 * 1024 * 1024),
    )(x, Wq, k_r, v_r, Wo)


# device time: 40222 ns/iter; 8.4383x vs baseline; 1.0626x over previous
import functools

import jax
import jax.numpy as jnp
from jax import lax
from jax.experimental import pallas as pl
from jax.experimental.pallas import tpu as pltpu

N_DEV = 4
WINDOW = 128
NGLOB = 32
HALO = 128
SCALE = 0.125
BF16 = jnp.bfloat16


def kernel(x, Wq, K_ext, V_ext, Wo):
    B, S_loc, E = x.shape
    _, _, HQ, DH = K_ext.shape
    DM = HQ * DH

    k_r = K_ext.reshape(B, S_loc, DM)
    v_r = V_ext.reshape(B, S_loc, DM)

    def body(x_ref, wq_ref, k_ref, v_ref, wo_ref, out_ref,
             kA, kB, vA, vB, klh, krh, vlh, vrh, kg, vg, xg, psend, pbuf,
             hs, hr, xs, xr, gks, gvs, gkr, gvr, pss, pr):
        my = lax.axis_index("i")
        left = lax.rem(my + N_DEV - 1, N_DEV)
        right = lax.rem(my + 1, N_DEV)
        opp = lax.rem(my + 2, N_DEV)

        barrier_sem = pltpu.get_barrier_semaphore()
        for nbr in (left, right, opp):
            pl.semaphore_signal(barrier_sem, inc=1, device_id=(nbr,),
                                device_id_type=pl.DeviceIdType.MESH)
        pl.semaphore_wait(barrier_sem, N_DEV - 1)

        @pl.when(my == 0)
        def _():
            xg[...] = x_ref[:, pl.ds(0, NGLOB)].astype(BF16)
            for t in range(1, N_DEV):
                pltpu.make_async_remote_copy(
                    src_ref=xg, dst_ref=xg, send_sem=xs.at[t - 1],
                    recv_sem=xr, device_id=(t,),
                    device_id_type=pl.DeviceIdType.MESH).start()

        kA[...] = k_ref[:, pl.ds(0, HALO)].astype(BF16)
        kB[...] = k_ref[:, pl.ds(S_loc - HALO, HALO)].astype(BF16)
        vA[...] = v_ref[:, pl.ds(0, HALO)].astype(BF16)
        vB[...] = v_ref[:, pl.ds(S_loc - HALO, HALO)].astype(BF16)

        halo_rdmas = []
        for i, (src, dst, tgt) in enumerate(
                ((kA, krh, left), (kB, klh, right),
                 (vA, vrh, left), (vB, vlh, right))):
            r = pltpu.make_async_remote_copy(
                src_ref=src, dst_ref=dst, send_sem=hs.at[i], recv_sem=hr.at[i],
                device_id=(tgt,), device_id_type=pl.DeviceIdType.MESH)
            r.start()
            halo_rdmas.append(r)

        @pl.when(my == 0)
        def _():
            for t in range(1, N_DEV):
                pltpu.make_async_remote_copy(
                    src_ref=kA.at[:, pl.ds(0, NGLOB)], dst_ref=kg,
                    send_sem=gks.at[t - 1], recv_sem=gkr, device_id=(t,),
                    device_id_type=pl.DeviceIdType.MESH).start()
                pltpu.make_async_remote_copy(
                    src_ref=vA.at[:, pl.ds(0, NGLOB)], dst_ref=vg,
                    send_sem=gvs.at[t - 1], recv_sem=gvr, device_id=(t,),
                    device_id_type=pl.DeviceIdType.MESH).start()
            kg[...] = jnp.zeros_like(kg)
            vg[...] = jnp.zeros_like(vg)

        pbuf[...] = jnp.zeros_like(pbuf)

        wq16 = wq_ref[...].astype(BF16)
        Qb = [jnp.dot(x_ref[b].astype(BF16), wq16,
                      preferred_element_type=jnp.float32) for b in range(B)]
        k16 = [k_ref[b].astype(BF16) for b in range(B)]
        v16 = [v_ref[b].astype(BF16) for b in range(B)]
        ones_own = jnp.ones((S_loc, 1), BF16)

        def hsl(hh):
            return slice(hh * DH, (hh + 1) * DH)

        @pl.when(my != 0)
        def _():
            pltpu.make_async_remote_copy(
                src_ref=xg, dst_ref=xg, send_sem=xs.at[0], recv_sem=xr,
                device_id=(0,), device_id_type=pl.DeviceIdType.MESH
            ).wait_recv()
            q32 = [jnp.dot(xg[b], wq16, preferred_element_type=jnp.float32)
                   for b in range(B)]
            kj_loc = my * S_loc + lax.broadcasted_iota(
                jnp.int32, (NGLOB, S_loc), 1)
            pmask = kj_loc >= (S_loc + HALO)
            for b in range(B):
                for hh in range(HQ):
                    qp = q32[b][:, hsl(hh)].astype(BF16)
                    sp = lax.dot_general(
                        qp, k16[b][:, hsl(hh)], (((1,), (1,)), ((), ())),
                        preferred_element_type=jnp.float32) * SCALE
                    wp = jnp.where(pmask, jnp.exp(sp), 0.0)
                    lp = jnp.sum(wp, axis=1, keepdims=True)
                    accp = jnp.dot(wp.astype(BF16), v16[b][:, hsl(hh)],
                                   preferred_element_type=jnp.float32)
                    psend[b, hh] = jnp.concatenate(
                        [accp, jnp.broadcast_to(lp, (NGLOB, DH))], axis=1)

        for d in range(1, N_DEV):
            @pl.when(my == d)
            def _(d=d):
                pltpu.make_async_remote_copy(
                    src_ref=psend, dst_ref=pbuf.at[d], send_sem=pss,
                    recv_sem=pr.at[d], device_id=(0,),
                    device_id_type=pl.DeviceIdType.MESH).start()

        def orig_mask(qi, kj):
            return (jnp.abs(qi - kj) <= WINDOW) | (kj < NGLOB) | (qi < NGLOB)

        def qi_col(w):
            return my * S_loc + lax.broadcasted_iota(jnp.int32, (S_loc, w), 0)

        kj_own = my * S_loc + lax.broadcasted_iota(
            jnp.int32, (S_loc, S_loc), 1)
        mask_own = orig_mask(qi_col(S_loc), kj_own)
        kj_l = my * S_loc - HALO + lax.broadcasted_iota(
            jnp.int32, (S_loc, HALO), 1)
        kj_r = (my + 1) * S_loc + lax.broadcasted_iota(
            jnp.int32, (S_loc, HALO), 1)
        kj_g = lax.broadcasted_iota(jnp.int32, (S_loc, NGLOB), 1)
        mask_h = jnp.concatenate([
            orig_mask(qi_col(HALO), kj_l) & (my > 0),
            orig_mask(qi_col(HALO), kj_r) & (my < N_DEV - 1),
            orig_mask(qi_col(NGLOB), kj_g) & (my != 0),
        ], axis=1)

        q16 = [[Qb[b][:, hsl(hh)].astype(BF16)
                for hh in range(HQ)] for b in range(B)]
        accA = [[None] * HQ for _ in range(B)]
        lA = [[None] * HQ for _ in range(B)]
        for b in range(B):
            for hh in range(HQ):
                sc = lax.dot_general(
                    q16[b][hh], k16[b][:, hsl(hh)], (((1,), (1,)), ((), ())),
                    preferred_element_type=jnp.float32) * SCALE
                w = jnp.exp(jnp.where(mask_own, sc, -1e9).astype(BF16))
                v_aug = jnp.concatenate(
                    [v16[b][:, hsl(hh)], ones_own], axis=1)
                aug = jnp.dot(w, v_aug,
                              preferred_element_type=jnp.float32)
                accA[b][hh] = aug[:, 0:DH]
                lA[b][hh] = aug[:, DH:DH + 1]

        halo_rdmas[0].wait_recv()
        halo_rdmas[1].wait_recv()

        @pl.when(my != 0)
        def _():
            pltpu.make_async_remote_copy(
                src_ref=kg, dst_ref=kg, send_sem=gks.at[0], recv_sem=gkr,
                device_id=(0,), device_id_type=pl.DeviceIdType.MESH
            ).wait_recv()

        wB = [[None] * HQ for _ in range(B)]
        for b in range(B):
            klh_b, krh_b, kg_b = klh[b], krh[b], kg[b]
            for hh in range(HQ):
                k_hcat = jnp.concatenate(
                    [klh_b[:, hsl(hh)], krh_b[:, hsl(hh)], kg_b[:, hsl(hh)]],
                    axis=0)
                sc = lax.dot_general(
                    q16[b][hh], k_hcat, (((1,), (1,)), ((), ())),
                    preferred_element_type=jnp.float32) * SCALE
                wB[b][hh] = jnp.exp(
                    jnp.where(mask_h, sc, -1e9).astype(BF16))

        halo_rdmas[2].wait_recv()
        halo_rdmas[3].wait_recv()

        @pl.when(my != 0)
        def _():
            pltpu.make_async_remote_copy(
                src_ref=vg, dst_ref=vg, send_sem=gvs.at[0], recv_sem=gvr,
                device_id=(0,), device_id_type=pl.DeviceIdType.MESH
            ).wait_recv()

        @pl.when(my == 0)
        def _():
            for d in range(1, N_DEV):
                pltpu.make_async_remote_copy(
                    src_ref=psend, dst_ref=pbuf.at[d], send_sem=pss,
                    recv_sem=pr.at[d], device_id=(0,),
                    device_id_type=pl.DeviceIdType.MESH).wait_recv()

        wo16 = wo_ref[...].astype(BF16)
        ones_h = jnp.ones((2 * HALO + NGLOB, 1), BF16)
        for b in range(B):
            vlh_b, vrh_b, vg_b = vlh[b], vrh[b], vg[b]
            ctx_cols = []
            for hh in range(HQ):
                v_hcat = jnp.concatenate(
                    [vlh_b[:, hsl(hh)], vrh_b[:, hsl(hh)], vg_b[:, hsl(hh)]],
                    axis=0)
                v_haug = jnp.concatenate([v_hcat, ones_h], axis=1)
                aug = jnp.dot(wB[b][hh], v_haug,
                              preferred_element_type=jnp.float32)
                l = lA[b][hh] + aug[:, DH:DH + 1]
                acc = accA[b][hh] + aug[:, 0:DH]
                p_acc = sum(pbuf[d, b, hh, :, 0:DH] for d in range(1, N_DEV))
                p_l = sum(pbuf[d, b, hh, :, DH:DH + 1]
                          for d in range(1, N_DEV))
                ctx = jnp.concatenate([
                    (acc[0:NGLOB] + p_acc) / (l[0:NGLOB] + p_l),
                    acc[NGLOB:] / l[NGLOB:],
                ], axis=0)
                ctx_cols.append(ctx)
            ctx_b = jnp.concatenate(ctx_cols, axis=1)
            out_ref[b] = jnp.dot(ctx_b.astype(BF16), wo16,
                                 preferred_element_type=jnp.float32)

        for r in halo_rdmas:
            r.wait_send()

        @pl.when(my == 0)
        def _():
            for t in range(1, N_DEV):
                pltpu.make_async_remote_copy(
                    src_ref=xg, dst_ref=xg, send_sem=xs.at[t - 1],
                    recv_sem=xr, device_id=(t,),
                    device_id_type=pl.DeviceIdType.MESH).wait_send()
                pltpu.make_async_remote_copy(
                    src_ref=kA.at[:, pl.ds(0, NGLOB)], dst_ref=kg,
                    send_sem=gks.at[t - 1], recv_sem=gkr, device_id=(t,),
                    device_id_type=pl.DeviceIdType.MESH).wait_send()
                pltpu.make_async_remote_copy(
                    src_ref=vA.at[:, pl.ds(0, NGLOB)], dst_ref=vg,
                    send_sem=gvs.at[t - 1], recv_sem=gvr, device_id=(t,),
                    device_id_type=pl.DeviceIdType.MESH).wait_send()

        @pl.when(my != 0)
        def _():
            pltpu.make_async_remote_copy(
                src_ref=psend, dst_ref=pbuf.at[1], send_sem=pss,
                recv_sem=pr.at[1], device_id=(0,),
                device_id_type=pl.DeviceIdType.MESH).wait_send()

        @functools.partial(pl.run_scoped,
                           second_barrier=pltpu.SemaphoreType.REGULAR)
        def _(second_barrier):
            for nbr in (left, right, opp):
                pl.semaphore_signal(second_barrier, inc=1, device_id=(nbr,),
                                    device_id_type=pl.DeviceIdType.MESH)
            pl.semaphore_wait(second_barrier, N_DEV - 1)

    return pl.pallas_call(
        body,
        out_shape=jax.ShapeDtypeStruct((B, S_loc, E), jnp.float32),
        in_specs=[pl.BlockSpec(memory_space=pltpu.VMEM)] * 5,
        out_specs=pl.BlockSpec(memory_space=pltpu.VMEM),
        scratch_shapes=[
            pltpu.VMEM((B, HALO, DM), BF16),
            pltpu.VMEM((B, HALO, DM), BF16),
            pltpu.VMEM((B, HALO, DM), BF16),
            pltpu.VMEM((B, HALO, DM), BF16),
            pltpu.VMEM((B, HALO, DM), BF16),
            pltpu.VMEM((B, HALO, DM), BF16),
            pltpu.VMEM((B, HALO, DM), BF16),
            pltpu.VMEM((B, HALO, DM), BF16),
            pltpu.VMEM((B, NGLOB, DM), BF16),
            pltpu.VMEM((B, NGLOB, DM), BF16),
            pltpu.VMEM((B, NGLOB, E), BF16),
            pltpu.VMEM((B, HQ, NGLOB, 2 * DH), jnp.float32),
            pltpu.VMEM((N_DEV, B, HQ, NGLOB, 2 * DH), jnp.float32),
            pltpu.SemaphoreType.DMA((4,)),
            pltpu.SemaphoreType.DMA((4,)),
            pltpu.SemaphoreType.DMA((3,)),
            pltpu.SemaphoreType.DMA,
            pltpu.SemaphoreType.DMA((3,)),
            pltpu.SemaphoreType.DMA((3,)),
            pltpu.SemaphoreType.DMA,
            pltpu.SemaphoreType.DMA,
            pltpu.SemaphoreType.DMA,
            pltpu.SemaphoreType.DMA((N_DEV,)),
        ],
        compiler_params=pltpu.CompilerParams(
            collective_id=0, vmem_limit_bytes=100 * 1024 * 1024),
    )(x, Wq, k_r, v_r, Wo)
